# Initial kernel scaffold; baseline (speedup 1.0000x reference)
import jax
import jax.numpy as jnp
from jax.experimental import pallas as pl


def kernel(x, edge_index, batch, W_gcn, b_gcn, W_cls, b_cls):
    raise NotImplementedError("write your pallas kernel here")


# speedup vs baseline: 36.5455x; 36.5455x over previous
import functools

import jax
import jax.numpy as jnp
from jax import lax
from jax.experimental import pallas as pl
from jax.experimental.pallas import tpu as pltpu
from jax.experimental.pallas import tpu_sc as plsc

N = 10000
E = 320000
D_IN = 128
D_OUT = 32
G = 64

NPAD = 10240
NC = 2
NS = 16
NW = NC * NS
EPW = E // NW
CHUNK = 128
NCHUNK = (EPW + CHUNK - 1) // CHUNK
EPW_PAD = NCHUNK * CHUNK
RPT = NPAD // NS

_mesh = plsc.VectorSubcoreMesh(core_axis_name="c", subcore_axis_name="s")


def _deg_body(dst_hbm, out_hbm, idx_v, ones_v, zeros_v, acc_sh):
    c = lax.axis_index("c")
    s = lax.axis_index("s")
    wid = c * NS + s

    def _fill_zero(i, carry):
        zeros_v[pl.ds(i * 16, 16)] = jnp.zeros((16,), jnp.float32)
        return carry

    lax.fori_loop(0, RPT // 16, _fill_zero, 0)

    def _fill_one(i, carry):
        ones_v[pl.ds(i * 16, 16)] = jnp.ones((16,), jnp.float32)
        return carry

    lax.fori_loop(0, CHUNK // 16, _fill_one, 0)

    pltpu.sync_copy(zeros_v, acc_sh.at[pl.ds(s * RPT, RPT)])
    plsc.subcore_barrier()

    pltpu.sync_copy(dst_hbm.at[wid], idx_v)

    def _chunk(j, carry):
        pltpu.sync_copy(ones_v, acc_sh.at[idx_v.at[j]], add=True)
        return carry

    lax.fori_loop(0, NCHUNK, _chunk, 0)
    plsc.subcore_barrier()
    pltpu.sync_copy(acc_sh.at[pl.ds(s * RPT, RPT)],
                    out_hbm.at[c, pl.ds(s * RPT, RPT)])


_deg_call = pl.kernel(
    _deg_body,
    out_type=jax.ShapeDtypeStruct((NC, NPAD), jnp.float32),
    mesh=_mesh,
    scratch_types=[
        pltpu.VMEM((NCHUNK, CHUNK), jnp.int32),
        pltpu.VMEM((CHUNK,), jnp.float32),
        pltpu.VMEM((RPT,), jnp.float32),
        pltpu.VMEM_SHARED((NPAD,), jnp.float32),
    ],
)


_ZR = 64


def _scat_body(z_hbm, src_hbm, dst_hbm, out_hbm,
               idxs_v, idxd_v, rows_v, zrow_v, acc_sh, sem):
    c = lax.axis_index("c")
    s = lax.axis_index("s")
    wid = c * NS + s

    def _fill_zero(r, carry):
        zrow_v[r, pl.ds(0, 16)] = jnp.zeros((16,), jnp.float32)
        zrow_v[r, pl.ds(16, 16)] = jnp.zeros((16,), jnp.float32)
        return carry

    lax.fori_loop(0, _ZR, _fill_zero, 0)

    def _zero_acc(k, carry):
        pltpu.sync_copy(zrow_v, acc_sh.at[pl.ds(s * RPT + k * _ZR, _ZR)])
        return carry

    lax.fori_loop(0, RPT // _ZR, _zero_acc, 0)
    plsc.subcore_barrier()

    pltpu.sync_copy(src_hbm.at[wid], idxs_v)
    pltpu.sync_copy(dst_hbm.at[wid], idxd_v)

    def _chunk(j, carry):
        pltpu.async_copy(z_hbm.at[idxs_v.at[j]], rows_v, sem).wait()
        pltpu.sync_copy(rows_v, acc_sh.at[idxd_v.at[j]], add=True)
        return carry

    lax.fori_loop(0, NCHUNK, _chunk, 0)
    plsc.subcore_barrier()

    def _out(k, carry):
        pltpu.sync_copy(acc_sh.at[pl.ds(s * RPT + k * _ZR, _ZR)],
                        out_hbm.at[c, pl.ds(s * RPT + k * _ZR, _ZR)])
        return carry

    lax.fori_loop(0, RPT // _ZR, _out, 0)


_scat_call = pl.kernel(
    _scat_body,
    out_type=jax.ShapeDtypeStruct((NC, NPAD, D_OUT), jnp.float32),
    mesh=_mesh,
    compiler_params=pltpu.CompilerParams(use_tc_tiling_on_sc=False),
    scratch_types=[
        pltpu.VMEM((NCHUNK, CHUNK), jnp.int32),
        pltpu.VMEM((NCHUNK, CHUNK), jnp.int32),
        pltpu.VMEM((CHUNK, D_OUT), jnp.float32),
        pltpu.VMEM((_ZR, D_OUT), jnp.float32),
        pltpu.VMEM_SHARED((NPAD, D_OUT), jnp.float32),
        pltpu.SemaphoreType.DMA,
    ],
)


def _mm_body(x_ref, w_ref, y_ref):
    y = jnp.dot(x_ref[...], w_ref[...],
                preferred_element_type=jnp.float32,
                precision=lax.Precision.HIGHEST)
    y_ref[0:N, :] = y
    y_ref[N:NPAD, :] = jnp.zeros((NPAD - N, D_OUT), jnp.float32)


def _mm_call(x, w):
    return pl.pallas_call(
        _mm_body,
        out_shape=jax.ShapeDtypeStruct((NPAD, D_OUT), jnp.float32),
    )(x, w)


def _scale_body(y_ref, d0_ref, d1_ref, z_ref, db_ref):
    deg = d0_ref[...] + d1_ref[...] + 1.0
    dinv = lax.rsqrt(deg)
    db = jnp.broadcast_to(dinv, (NPAD, D_OUT))
    z_ref[...] = y_ref[...] * db
    db_ref[...] = db


def _scale_call(y, d0, d1):
    return pl.pallas_call(
        _scale_body,
        out_shape=[jax.ShapeDtypeStruct((NPAD, D_OUT), jnp.float32),
                   jax.ShapeDtypeStruct((NPAD, D_OUT), jnp.float32)],
    )(y, d0, d1)


RBLK = 1000
NBLK = N // RBLK
GD = G * D_OUT


def _pool_body(batch_ref, s0_ref, s1_ref, z_ref, db_ref, bg_ref, wt1_ref,
               wt2_ref, bc_ref, out_ref, macc, sacc, cacc):
    i = pl.program_id(0)

    @pl.when(i == 0)
    def _():
        macc[...] = jnp.full((1, GD), -jnp.inf, jnp.float32)
        sacc[...] = jnp.zeros((1, GD), jnp.float32)
        cacc[...] = jnp.zeros((1, GD), jnp.float32)

    agg = (s0_ref[...] + s1_ref[...] + z_ref[...]) * db_ref[...]
    h = jnp.maximum(agg + bg_ref[...], 0.0)
    ht = jnp.broadcast_to(h[:, None, :], (RBLK, G, D_OUT)).reshape(RBLK, GD)
    gcol = lax.broadcasted_iota(jnp.int32, (RBLK, GD), 1) // D_OUT
    mask = batch_ref[0] == gcol

    macc[...] = jnp.maximum(
        macc[...],
        jnp.max(jnp.where(mask, ht, -jnp.inf), axis=0, keepdims=True))
    sacc[...] = sacc[...] + jnp.sum(jnp.where(mask, ht, 0.0), axis=0,
                                    keepdims=True)
    cacc[...] = cacc[...] + jnp.sum(mask.astype(jnp.float32), axis=0,
                                    keepdims=True)

    @pl.when(i == NBLK - 1)
    def _():
        gap = sacc[...] / jnp.maximum(cacc[...], 1.0)
        gmp = jnp.maximum(macc[...], jnp.float32(-1e30))
        contrib = gap * wt1_ref[...] + gmp * wt2_ref[...]
        iog = lax.broadcasted_iota(jnp.int32, (G, GD), 0)
        ioc = lax.broadcasted_iota(jnp.int32, (G, GD), 1) // D_OUT
        sel = (iog == ioc).astype(jnp.float32)
        out = lax.dot_general(contrib, sel, (((1,), (1,)), ((), ())),
                              preferred_element_type=jnp.float32,
                              precision=lax.Precision.HIGHEST)
        out_ref[...] = out + bc_ref[...]


def _pool_call(batch3, s0, s1, z, db, bg, wt1, wt2, bc):
    row_spec = pl.BlockSpec((RBLK, D_OUT), lambda i: (i, 0))
    full = lambda shape: pl.BlockSpec(shape, lambda i: tuple(0 for _ in shape))
    return pl.pallas_call(
        _pool_body,
        grid=(NBLK,),
        in_specs=[
            pl.BlockSpec((1, RBLK, 1), lambda i: (i, 0, 0)),
            row_spec, row_spec, row_spec, row_spec,
            full((1, D_OUT)), full((1, GD)), full((1, GD)), full((1, G)),
        ],
        out_specs=full((1, G)),
        out_shape=jax.ShapeDtypeStruct((1, G), jnp.float32),
        scratch_shapes=[
            pltpu.VMEM((1, GD), jnp.float32),
            pltpu.VMEM((1, GD), jnp.float32),
            pltpu.VMEM((1, GD), jnp.float32),
        ],
    )(batch3, s0, s1, z, db, bg, wt1, wt2, bc)


def kernel(x, edge_index, batch, W_gcn, b_gcn, W_cls, b_cls):
    src = edge_index[0]
    dst = edge_index[1]

    pad_vals = N + (jnp.arange(EPW_PAD - EPW, dtype=jnp.int32) % (NPAD - N))

    def shard(a):
        a2 = a.reshape(NW, EPW)
        pad = jnp.broadcast_to(pad_vals[None, :], (NW, EPW_PAD - EPW))
        return jnp.concatenate([a2, pad], axis=1).reshape(NW, NCHUNK, CHUNK)

    src_sh = shard(src)
    dst_sh = shard(dst)

    degp = _deg_call(dst_sh)
    y = _mm_call(x, W_gcn)
    z, db = _scale_call(y, degp[0][:, None], degp[1][:, None])
    s = _scat_call(z, src_sh, dst_sh)

    batch3 = batch.reshape(NBLK, RBLK, 1)
    wt1 = jnp.tile(W_cls[0:D_OUT, 0], G).reshape(1, GD)
    wt2 = jnp.tile(W_cls[D_OUT:2 * D_OUT, 0], G).reshape(1, GD)
    out = _pool_call(batch3, s[0], s[1], z, db,
                     b_gcn.reshape(1, D_OUT), wt1, wt2,
                     jnp.broadcast_to(b_cls.reshape(1, 1), (1, G)))
    return out.reshape(G, 1)


# speedup vs baseline: 47.4668x; 1.2988x over previous
import functools

import jax
import jax.numpy as jnp
from jax import lax
from jax.experimental import pallas as pl
from jax.experimental.pallas import tpu as pltpu
from jax.experimental.pallas import tpu_sc as plsc

N = 10000
E = 320000
D_IN = 128
D_OUT = 32
G = 64

NPAD = 10240
NC = 2
NS = 16
NW = NC * NS
EPW = E // NW
CHUNK = 128
NBUF = 8
NCHUNK = 80
EPW_PAD = NCHUNK * CHUNK
RPT = NPAD // NS

_mesh = plsc.VectorSubcoreMesh(core_axis_name="c", subcore_axis_name="s")


def _deg_body(dst_hbm, out_hbm, idx_v, ones_v, zeros_v, acc_sh, sem):
    c = lax.axis_index("c")
    s = lax.axis_index("s")
    wid = c * NS + s

    def _fill_zero(i, carry):
        zeros_v[pl.ds(i * 16, 16)] = jnp.zeros((16,), jnp.float32)
        return carry

    lax.fori_loop(0, RPT // 16, _fill_zero, 0)

    def _fill_one(i, carry):
        ones_v[pl.ds(i * 16, 16)] = jnp.ones((16,), jnp.float32)
        return carry

    lax.fori_loop(0, CHUNK // 16, _fill_one, 0)

    pltpu.sync_copy(zeros_v, acc_sh.at[pl.ds(s * RPT, RPT)])
    plsc.subcore_barrier()

    pltpu.sync_copy(dst_hbm.at[wid], idx_v)

    def _chunk(r, carry):
        for b in range(NBUF):
            pltpu.async_copy(ones_v, acc_sh.at[idx_v.at[r * NBUF + b]],
                             sem, add=True)
        for b in range(NBUF):
            pltpu.make_async_copy(ones_v, acc_sh.at[idx_v.at[r * NBUF + b]],
                                  sem).wait()
        return carry

    lax.fori_loop(0, NCHUNK // NBUF, _chunk, 0)
    plsc.subcore_barrier()
    pltpu.sync_copy(acc_sh.at[pl.ds(s * RPT, RPT)],
                    out_hbm.at[c, pl.ds(s * RPT, RPT)])


_deg_call = pl.kernel(
    _deg_body,
    out_type=jax.ShapeDtypeStruct((NC, NPAD), jnp.float32),
    mesh=_mesh,
    scratch_types=[
        pltpu.VMEM((NCHUNK, CHUNK), jnp.int32),
        pltpu.VMEM((CHUNK,), jnp.float32),
        pltpu.VMEM((RPT,), jnp.float32),
        pltpu.VMEM_SHARED((NPAD,), jnp.float32),
        pltpu.SemaphoreType.DMA,
    ],
)


_ZR = 64


def _scat_body(z_hbm, src_hbm, dst_hbm, out_hbm,
               idxs_v, idxd_v, rows_v, zrow_v, acc_sh, sem_g, sem_s):
    c = lax.axis_index("c")
    s = lax.axis_index("s")
    wid = c * NS + s

    def _fill_zero(r, carry):
        zrow_v[r, pl.ds(0, 16)] = jnp.zeros((16,), jnp.float32)
        zrow_v[r, pl.ds(16, 16)] = jnp.zeros((16,), jnp.float32)
        return carry

    lax.fori_loop(0, _ZR, _fill_zero, 0)

    def _zero_acc(k, carry):
        pltpu.sync_copy(zrow_v, acc_sh.at[pl.ds(s * RPT + k * _ZR, _ZR)])
        return carry

    lax.fori_loop(0, RPT // _ZR, _zero_acc, 0)
    plsc.subcore_barrier()

    pltpu.sync_copy(src_hbm.at[wid], idxs_v)
    pltpu.sync_copy(dst_hbm.at[wid], idxd_v)

    for b in range(NBUF):
        pltpu.async_copy(z_hbm.at[idxs_v.at[b]], rows_v.at[0, b], sem_g)

    def _round(sr, carry):
        base = sr * 2 * NBUF
        for st in (0, 1):
            cur = base + st * NBUF
            nxt = cur + NBUF
            for b in range(NBUF):
                pltpu.make_async_copy(z_hbm.at[idxs_v.at[cur + b]],
                                      rows_v.at[st, b], sem_g).wait()
            for b in range(NBUF):
                pltpu.async_copy(rows_v.at[st, b],
                                 acc_sh.at[idxd_v.at[cur + b]],
                                 sem_s, add=True)

            @pl.when(nxt < NCHUNK)
            def _():
                for b in range(NBUF):
                    pltpu.async_copy(z_hbm.at[idxs_v.at[nxt + b]],
                                     rows_v.at[1 - st, b], sem_g)

            for b in range(NBUF):
                pltpu.make_async_copy(rows_v.at[st, b],
                                      acc_sh.at[idxd_v.at[cur + b]],
                                      sem_s).wait()
        return carry

    lax.fori_loop(0, NCHUNK // (2 * NBUF), _round, 0)
    plsc.subcore_barrier()

    def _out(k, carry):
        pltpu.sync_copy(acc_sh.at[pl.ds(s * RPT + k * _ZR, _ZR)],
                        out_hbm.at[c, pl.ds(s * RPT + k * _ZR, _ZR)])
        return carry

    lax.fori_loop(0, RPT // _ZR, _out, 0)


_scat_call = pl.kernel(
    _scat_body,
    out_type=jax.ShapeDtypeStruct((NC, NPAD, D_OUT), jnp.float32),
    mesh=_mesh,
    compiler_params=pltpu.CompilerParams(use_tc_tiling_on_sc=False),
    scratch_types=[
        pltpu.VMEM((NCHUNK, CHUNK), jnp.int32),
        pltpu.VMEM((NCHUNK, CHUNK), jnp.int32),
        pltpu.VMEM((2, NBUF, CHUNK, D_OUT), jnp.float32),
        pltpu.VMEM((_ZR, D_OUT), jnp.float32),
        pltpu.VMEM_SHARED((NPAD, D_OUT), jnp.float32),
        pltpu.SemaphoreType.DMA,
        pltpu.SemaphoreType.DMA,
    ],
)


def _mm_body(x_ref, w_ref, y_ref):
    y = jnp.dot(x_ref[...], w_ref[...],
                preferred_element_type=jnp.float32,
                precision=lax.Precision.HIGHEST)
    y_ref[0:N, :] = y
    y_ref[N:NPAD, :] = jnp.zeros((NPAD - N, D_OUT), jnp.float32)


def _mm_call(x, w):
    return pl.pallas_call(
        _mm_body,
        out_shape=jax.ShapeDtypeStruct((NPAD, D_OUT), jnp.float32),
    )(x, w)


def _scale_body(y_ref, d0_ref, d1_ref, z_ref, db_ref):
    deg = d0_ref[...] + d1_ref[...] + 1.0
    dinv = lax.rsqrt(deg)
    db = jnp.broadcast_to(dinv, (NPAD, D_OUT))
    z_ref[...] = y_ref[...] * db
    db_ref[...] = db


def _scale_call(y, d0, d1):
    return pl.pallas_call(
        _scale_body,
        out_shape=[jax.ShapeDtypeStruct((NPAD, D_OUT), jnp.float32),
                   jax.ShapeDtypeStruct((NPAD, D_OUT), jnp.float32)],
    )(y, d0, d1)


RBLK = 1000
NBLK = N // RBLK
GD = G * D_OUT


def _pool_body(batch_ref, s0_ref, s1_ref, z_ref, db_ref, bg_ref, wt1_ref,
               wt2_ref, bc_ref, out_ref, macc, sacc, cacc):
    i = pl.program_id(0)

    @pl.when(i == 0)
    def _():
        macc[...] = jnp.full((1, GD), -jnp.inf, jnp.float32)
        sacc[...] = jnp.zeros((1, GD), jnp.float32)
        cacc[...] = jnp.zeros((1, GD), jnp.float32)

    agg = (s0_ref[...] + s1_ref[...] + z_ref[...]) * db_ref[...]
    h = jnp.maximum(agg + bg_ref[...], 0.0)
    ht = jnp.broadcast_to(h[:, None, :], (RBLK, G, D_OUT)).reshape(RBLK, GD)
    gcol = lax.broadcasted_iota(jnp.int32, (RBLK, GD), 1) // D_OUT
    mask = batch_ref[0] == gcol

    macc[...] = jnp.maximum(
        macc[...],
        jnp.max(jnp.where(mask, ht, -jnp.inf), axis=0, keepdims=True))
    sacc[...] = sacc[...] + jnp.sum(jnp.where(mask, ht, 0.0), axis=0,
                                    keepdims=True)
    cacc[...] = cacc[...] + jnp.sum(mask.astype(jnp.float32), axis=0,
                                    keepdims=True)

    @pl.when(i == NBLK - 1)
    def _():
        gap = sacc[...] / jnp.maximum(cacc[...], 1.0)
        gmp = jnp.maximum(macc[...], jnp.float32(-1e30))
        contrib = gap * wt1_ref[...] + gmp * wt2_ref[...]
        iog = lax.broadcasted_iota(jnp.int32, (G, GD), 0)
        ioc = lax.broadcasted_iota(jnp.int32, (G, GD), 1) // D_OUT
        sel = (iog == ioc).astype(jnp.float32)
        out = lax.dot_general(contrib, sel, (((1,), (1,)), ((), ())),
                              preferred_element_type=jnp.float32,
                              precision=lax.Precision.HIGHEST)
        out_ref[...] = out + bc_ref[...]


def _pool_call(batch3, s0, s1, z, db, bg, wt1, wt2, bc):
    row_spec = pl.BlockSpec((RBLK, D_OUT), lambda i: (i, 0))
    full = lambda shape: pl.BlockSpec(shape, lambda i: tuple(0 for _ in shape))
    return pl.pallas_call(
        _pool_body,
        grid=(NBLK,),
        in_specs=[
            pl.BlockSpec((1, RBLK, 1), lambda i: (i, 0, 0)),
            row_spec, row_spec, row_spec, row_spec,
            full((1, D_OUT)), full((1, GD)), full((1, GD)), full((1, G)),
        ],
        out_specs=full((1, G)),
        out_shape=jax.ShapeDtypeStruct((1, G), jnp.float32),
        scratch_shapes=[
            pltpu.VMEM((1, GD), jnp.float32),
            pltpu.VMEM((1, GD), jnp.float32),
            pltpu.VMEM((1, GD), jnp.float32),
        ],
    )(batch3, s0, s1, z, db, bg, wt1, wt2, bc)


def kernel(x, edge_index, batch, W_gcn, b_gcn, W_cls, b_cls):
    src = edge_index[0]
    dst = edge_index[1]

    pad_vals = N + (jnp.arange(EPW_PAD - EPW, dtype=jnp.int32) % (NPAD - N))

    def shard(a):
        a2 = a.reshape(NW, EPW)
        pad = jnp.broadcast_to(pad_vals[None, :], (NW, EPW_PAD - EPW))
        return jnp.concatenate([a2, pad], axis=1).reshape(NW, NCHUNK, CHUNK)

    src_sh = shard(src)
    dst_sh = shard(dst)

    degp = _deg_call(dst_sh)
    y = _mm_call(x, W_gcn)
    z, db = _scale_call(y, degp[0][:, None], degp[1][:, None])
    s = _scat_call(z, src_sh, dst_sh)

    batch3 = batch.reshape(NBLK, RBLK, 1)
    wt1 = jnp.tile(W_cls[0:D_OUT, 0], G).reshape(1, GD)
    wt2 = jnp.tile(W_cls[D_OUT:2 * D_OUT, 0], G).reshape(1, GD)
    out = _pool_call(batch3, s[0], s[1], z, db,
                     b_gcn.reshape(1, D_OUT), wt1, wt2,
                     jnp.broadcast_to(b_cls.reshape(1, 1), (1, G)))
    return out.reshape(G, 1)


# speedup vs baseline: 55.7219x; 1.1739x over previous
import functools

import jax
import jax.numpy as jnp
from jax import lax
from jax.experimental import pallas as pl
from jax.experimental.pallas import tpu as pltpu
from jax.experimental.pallas import tpu_sc as plsc

N = 10000
E = 320000
D_IN = 128
D_OUT = 32
G = 64

NPAD = 10240
NC = 2
NS = 16
NW = NC * NS
EPW = E // NW
CHUNK = 128
NBUF = 8
NCHUNK = 80
EPW_PAD = NCHUNK * CHUNK
RPT = NPAD // NS

_mesh = plsc.VectorSubcoreMesh(core_axis_name="c", subcore_axis_name="s")


def _deg_body(ei_hbm, out_hbm, idx_v, ones_v, zeros_v, acc_sh, sem):
    c = lax.axis_index("c")
    s = lax.axis_index("s")
    wid = c * NS + s

    def _fill_zero(i, carry):
        zeros_v[pl.ds(i * 16, 16)] = jnp.zeros((16,), jnp.float32)
        return carry

    lax.fori_loop(0, RPT // 16, _fill_zero, 0)

    def _fill_one(i, carry):
        ones_v[pl.ds(i * 16, 16)] = jnp.ones((16,), jnp.float32)
        return carry

    lax.fori_loop(0, CHUNK // 16, _fill_one, 0)

    pltpu.sync_copy(zeros_v, acc_sh.at[pl.ds(s * RPT, RPT)])
    plsc.subcore_barrier()

    pltpu.sync_copy(ei_hbm.at[1, wid], idx_v)

    def _chunk(r, carry):
        for b in range(NBUF):
            pltpu.async_copy(ones_v, acc_sh.at[idx_v.at[r * NBUF + b]],
                             sem, add=True)
        for b in range(NBUF):
            pltpu.make_async_copy(ones_v, acc_sh.at[idx_v.at[r * NBUF + b]],
                                  sem).wait()
        return carry

    lax.fori_loop(0, NCHUNK // NBUF, _chunk, 0)
    plsc.subcore_barrier()
    pltpu.sync_copy(acc_sh.at[pl.ds(s * RPT, RPT)],
                    out_hbm.at[c, pl.ds(s * RPT, RPT)])


_deg_call = pl.kernel(
    _deg_body,
    out_type=jax.ShapeDtypeStruct((NC, NPAD), jnp.float32),
    mesh=_mesh,
    scratch_types=[
        pltpu.VMEM((NCHUNK, CHUNK), jnp.int32),
        pltpu.VMEM((CHUNK,), jnp.float32),
        pltpu.VMEM((RPT,), jnp.float32),
        pltpu.VMEM_SHARED((NPAD,), jnp.float32),
        pltpu.SemaphoreType.DMA,
    ],
)


_ZR = 64


def _scat_body(z_hbm, ei_hbm, out_hbm,
               idxs_v, idxd_v, rows_v, zrow_v, acc_sh, sem_g, sem_s):
    c = lax.axis_index("c")
    s = lax.axis_index("s")
    wid = c * NS + s

    def _fill_zero(r, carry):
        zrow_v[r, pl.ds(0, 16)] = jnp.zeros((16,), jnp.float32)
        zrow_v[r, pl.ds(16, 16)] = jnp.zeros((16,), jnp.float32)
        return carry

    lax.fori_loop(0, _ZR, _fill_zero, 0)

    def _zero_acc(k, carry):
        pltpu.sync_copy(zrow_v, acc_sh.at[pl.ds(s * RPT + k * _ZR, _ZR)])
        return carry

    lax.fori_loop(0, RPT // _ZR, _zero_acc, 0)
    plsc.subcore_barrier()

    pltpu.sync_copy(ei_hbm.at[0, wid], idxs_v)
    pltpu.sync_copy(ei_hbm.at[1, wid], idxd_v)

    for b in range(NBUF):
        pltpu.async_copy(z_hbm.at[idxs_v.at[b]], rows_v.at[0, b], sem_g)

    def _round(sr, carry):
        base = sr * 2 * NBUF
        for st in (0, 1):
            cur = base + st * NBUF
            nxt = cur + NBUF
            for b in range(NBUF):
                pltpu.make_async_copy(z_hbm.at[idxs_v.at[cur + b]],
                                      rows_v.at[st, b], sem_g).wait()
            for b in range(NBUF):
                pltpu.async_copy(rows_v.at[st, b],
                                 acc_sh.at[idxd_v.at[cur + b]],
                                 sem_s, add=True)

            @pl.when(nxt < NCHUNK)
            def _():
                for b in range(NBUF):
                    pltpu.async_copy(z_hbm.at[idxs_v.at[nxt + b]],
                                     rows_v.at[1 - st, b], sem_g)

            for b in range(NBUF):
                pltpu.make_async_copy(rows_v.at[st, b],
                                      acc_sh.at[idxd_v.at[cur + b]],
                                      sem_s).wait()
        return carry

    lax.fori_loop(0, NCHUNK // (2 * NBUF), _round, 0)
    plsc.subcore_barrier()

    def _out(k, carry):
        pltpu.sync_copy(acc_sh.at[pl.ds(s * RPT + k * _ZR, _ZR)],
                        out_hbm.at[c, pl.ds(s * RPT + k * _ZR, _ZR)])
        return carry

    lax.fori_loop(0, RPT // _ZR, _out, 0)


_scat_call = pl.kernel(
    _scat_body,
    out_type=jax.ShapeDtypeStruct((NC, NPAD, D_OUT), jnp.float32),
    mesh=_mesh,
    compiler_params=pltpu.CompilerParams(use_tc_tiling_on_sc=False),
    scratch_types=[
        pltpu.VMEM((NCHUNK, CHUNK), jnp.int32),
        pltpu.VMEM((NCHUNK, CHUNK), jnp.int32),
        pltpu.VMEM((2, NBUF, CHUNK, D_OUT), jnp.float32),
        pltpu.VMEM((_ZR, D_OUT), jnp.float32),
        pltpu.VMEM_SHARED((NPAD, D_OUT), jnp.float32),
        pltpu.SemaphoreType.DMA,
        pltpu.SemaphoreType.DMA,
    ],
)


def _mmscale_body(x_ref, w_ref, d0_ref, d1_ref, z_ref, db_ref):
    deg = d0_ref[...] + d1_ref[...] + 1.0
    dinv = lax.rsqrt(deg)
    db = jnp.broadcast_to(dinv, (NPAD, D_OUT))
    y = jnp.dot(x_ref[...], w_ref[...],
                preferred_element_type=jnp.float32,
                precision=lax.Precision.HIGHEST)
    z_ref[0:N, :] = y * db[0:N, :]
    z_ref[N:NPAD, :] = jnp.zeros((NPAD - N, D_OUT), jnp.float32)
    db_ref[...] = db


def _mmscale_call(x, w, d0, d1):
    return pl.pallas_call(
        _mmscale_body,
        out_shape=[jax.ShapeDtypeStruct((NPAD, D_OUT), jnp.float32),
                   jax.ShapeDtypeStruct((NPAD, D_OUT), jnp.float32)],
    )(x, w, d0, d1)


RBLK = 1000
NBLK = N // RBLK
GD = G * D_OUT


def _pool_body(batch_ref, s0_ref, s1_ref, z_ref, db_ref, bg_ref, w1_ref,
               wt2_ref, bc_ref, out_ref, macc, sacc, cacc):
    i = pl.program_id(0)

    @pl.when(i == 0)
    def _():
        macc[...] = jnp.full((1, GD), -jnp.inf, jnp.float32)
        sacc[...] = jnp.zeros((G, D_OUT), jnp.float32)
        cacc[...] = jnp.zeros((1, G), jnp.float32)

    agg = (s0_ref[...] + s1_ref[...] + z_ref[...]) * db_ref[...]
    h = jnp.maximum(agg + bg_ref[...], 0.0)

    onehot = (batch_ref[0] ==
              lax.broadcasted_iota(jnp.int32, (RBLK, G), 1)
              ).astype(jnp.float32)
    sacc[...] = sacc[...] + lax.dot_general(
        onehot, h, (((0,), (0,)), ((), ())),
        preferred_element_type=jnp.float32,
        precision=lax.Precision.HIGHEST)
    cacc[...] = cacc[...] + jnp.sum(onehot, axis=0, keepdims=True)

    ht = jnp.broadcast_to(h[:, None, :], (RBLK, G, D_OUT)).reshape(RBLK, GD)
    gcol = lax.broadcasted_iota(jnp.int32, (RBLK, GD), 1) // D_OUT
    mask = batch_ref[0] == gcol
    macc[...] = jnp.maximum(
        macc[...],
        jnp.max(jnp.where(mask, ht, -jnp.inf), axis=0, keepdims=True))

    @pl.when(i == NBLK - 1)
    def _():
        wnum = lax.dot_general(w1_ref[...], sacc[...], (((1,), (1,)), ((), ())),
                               preferred_element_type=jnp.float32,
                               precision=lax.Precision.HIGHEST)
        out_gap = wnum / jnp.maximum(cacc[...], 1.0)
        gmp = jnp.maximum(macc[...], jnp.float32(-1e30))
        iog = lax.broadcasted_iota(jnp.int32, (G, GD), 0)
        ioc = lax.broadcasted_iota(jnp.int32, (G, GD), 1) // D_OUT
        sel = (iog == ioc).astype(jnp.float32)
        out_max = lax.dot_general(gmp * wt2_ref[...], sel,
                                  (((1,), (1,)), ((), ())),
                                  preferred_element_type=jnp.float32,
                                  precision=lax.Precision.HIGHEST)
        out_ref[...] = out_gap + out_max + bc_ref[...]


def _pool_call(batch3, s0, s1, z, db, bg, w1, wt2, bc):
    row_spec = pl.BlockSpec((RBLK, D_OUT), lambda i: (i, 0))
    full = lambda shape: pl.BlockSpec(shape, lambda i: tuple(0 for _ in shape))
    return pl.pallas_call(
        _pool_body,
        grid=(NBLK,),
        in_specs=[
            pl.BlockSpec((1, RBLK, 1), lambda i: (i, 0, 0)),
            row_spec, row_spec, row_spec, row_spec,
            full((1, D_OUT)), full((1, D_OUT)), full((1, GD)), full((1, G)),
        ],
        out_specs=full((1, G)),
        out_shape=jax.ShapeDtypeStruct((1, G), jnp.float32),
        scratch_shapes=[
            pltpu.VMEM((1, GD), jnp.float32),
            pltpu.VMEM((G, D_OUT), jnp.float32),
            pltpu.VMEM((1, G), jnp.float32),
        ],
    )(batch3, s0, s1, z, db, bg, w1, wt2, bc)


def kernel(x, edge_index, batch, W_gcn, b_gcn, W_cls, b_cls):
    pad_vals = N + (jnp.arange(EPW_PAD - EPW, dtype=jnp.int32) % (NPAD - N))
    pad = jnp.broadcast_to(pad_vals[None, None, :], (2, NW, EPW_PAD - EPW))
    ei = jnp.concatenate([edge_index.reshape(2, NW, EPW), pad],
                         axis=2).reshape(2, NW, NCHUNK, CHUNK)

    degp = _deg_call(ei)
    z, db = _mmscale_call(x, W_gcn, degp[0][:, None], degp[1][:, None])
    s = _scat_call(z, ei)

    batch3 = batch.reshape(NBLK, RBLK, 1)
    w1 = W_cls[0:D_OUT, 0].reshape(1, D_OUT)
    wt2 = jnp.tile(W_cls[D_OUT:2 * D_OUT, 0], G).reshape(1, GD)
    out = _pool_call(batch3, s[0], s[1], z, db,
                     b_gcn.reshape(1, D_OUT), w1, wt2,
                     jnp.broadcast_to(b_cls.reshape(1, 1), (1, G)))
    return out.reshape(G, 1)
